# baseline (speedup 1.0000x reference)
import functools

import jax
import jax.numpy as jnp
import numpy as np
from jax.experimental import pallas as pl

_AVG_LOG = float(np.log(33.0))
_N = 10000
_E = 320000
_NG = 64
_BE = 1600


def _mm_body(x_ref, w_ref, b_ref, o_ref):
    o_ref[...] = x_ref[...] @ w_ref[...] + b_ref[...][None, :]


def _edge_mm(x, w, b):
    K = x.shape[1]
    F = w.shape[1]
    return pl.pallas_call(
        _mm_body,
        grid=(_E // _BE,),
        in_specs=[
            pl.BlockSpec((_BE, K), lambda i: (i, 0)),
            pl.BlockSpec((K, F), lambda i: (0, 0)),
            pl.BlockSpec((F,), lambda i: (0,)),
        ],
        out_specs=pl.BlockSpec((_BE, F), lambda i: (i, 0)),
        out_shape=jax.ShapeDtypeStruct((_E, F), jnp.float32),
    )(x, w, b)


def kernel(node_attr, edge_index, edge_attr, batch,
           We1, be1, Wpre1, bpre1, Wpost1, bpost1, Wlin1, blin1,
           We2, be2, Wpre2, bpre2, Wpost2, bpost2, Wlin2, blin2,
           Wd1, bd1, gamma, beta, Wd2, bd2):
    src, dst = edge_index[0], edge_index[1]
    deg = jax.ops.segment_sum(jnp.ones((_E,), jnp.float32), dst, num_segments=_N)
    degc = jnp.maximum(deg, 1.0)
    amp = jnp.log(degc + 1.0) / _AVG_LOG
    att = _AVG_LOG / jnp.log(degc + 1.0)
    has = (deg > 0)[:, None]

    def pna(h, We, be, Wpre, bpre, Wpost, bpost, Wlin, blin):
        ee = edge_attr @ We + be
        m = _edge_mm(jnp.concatenate([h[dst], h[src], ee], axis=-1), Wpre, bpre)
        s = jax.ops.segment_sum(m, dst, num_segments=_N)
        mean = s / degc[:, None]
        mn = jnp.where(has, jax.ops.segment_min(m, dst, num_segments=_N), 0.0)
        mx = jnp.where(has, jax.ops.segment_max(m, dst, num_segments=_N), 0.0)
        mean2 = jax.ops.segment_sum(m * m, dst, num_segments=_N) / degc[:, None]
        std = jnp.sqrt(jnp.maximum(mean2 - mean * mean, 0.0) + 1e-5)
        agg = jnp.concatenate([mean, mn, mx, std], axis=-1)
        sc = jnp.concatenate([agg, agg * amp[:, None], agg * att[:, None]], axis=-1)
        o = jnp.concatenate([h, sc], axis=-1) @ Wpost + bpost
        return o @ Wlin + blin

    h = jax.nn.relu(pna(node_attr, We1, be1, Wpre1, bpre1, Wpost1, bpost1, Wlin1, blin1))
    h = jax.nn.relu(pna(h, We2, be2, Wpre2, bpre2, Wpost2, bpost2, Wlin2, blin2))
    cnt = jnp.maximum(jax.ops.segment_sum(jnp.ones((_N,), jnp.float32), batch, num_segments=_NG), 1.0)
    g = jax.ops.segment_sum(h, batch, num_segments=_NG) / cnt[:, None]
    z = g @ Wd1 + bd1
    z = (z - z.mean(0)) / jnp.sqrt(z.var(0) + 1e-5) * gamma + beta
    z = jnp.where(z > 0, z, 0.1 * z)
    return z @ Wd2 + bd2


# speedup vs baseline: 1.3080x; 1.3080x over previous
import functools

import jax
import jax.numpy as jnp
import numpy as np
from jax import lax
from jax.experimental import pallas as pl
from jax.experimental.pallas import tpu as pltpu
from jax.experimental.pallas import tpu_sc as plsc

_AVG_LOG = float(np.log(33.0))
_N = 10000
_E = 320000
_NG = 64
_NW = 32
_EC = _E // _NW
_CH = 2000
_GCH = 200
_BE = 1600
_BN = 400

_mesh = plsc.VectorSubcoreMesh(core_axis_name="c", subcore_axis_name="s")


def _wid():
    return lax.axis_index("s") * 2 + lax.axis_index("c")


def _make_g(F, NR):
    @functools.partial(
        pl.kernel, mesh=_mesh,
        out_type=jax.ShapeDtypeStruct((_E, F), jnp.float32),
        scratch_types=[
            pltpu.VMEM((_GCH,), jnp.int32),
            pltpu.VMEM((_GCH, F), jnp.float32),
            pltpu.SemaphoreType.DMA,
        ],
    )
    def g(idx_hbm, tab_hbm, out_hbm, ibuf, rows, sem):
        w = _wid()
        base = w * _EC

        def chunk(c, _):
            cb = pl.multiple_of(base + c * _GCH, 8)
            pltpu.sync_copy(idx_hbm.at[pl.ds(cb, _GCH)], ibuf)
            pltpu.async_copy(tab_hbm.at[ibuf], rows, sem).wait()
            pltpu.sync_copy(rows, out_hbm.at[pl.ds(cb, _GCH)])
            return 0
        lax.fori_loop(0, _EC // _GCH, chunk, 0)
    return g


_g128n = _make_g(128, _N)
_g256n = _make_g(256, _N)


def _mm_body(x_ref, w_ref, b_ref, o_ref):
    o_ref[...] = x_ref[...] @ w_ref[...] + b_ref[...][None, :]


def _edge_mm(x, w, b):
    K = x.shape[1]
    F = w.shape[1]
    return pl.pallas_call(
        _mm_body,
        grid=(_E // _BE,),
        in_specs=[
            pl.BlockSpec((_BE, K), lambda i: (i, 0)),
            pl.BlockSpec((K, F), lambda i: (0, 0)),
            pl.BlockSpec((F,), lambda i: (0,)),
        ],
        out_specs=pl.BlockSpec((_BE, F), lambda i: (i, 0)),
        out_shape=jax.ShapeDtypeStruct((_E, F), jnp.float32),
    )(x, w, b)


def _msg_body(xd_ref, xs_ref, ee_ref, w_ref, b_ref, o_ref):
    x = jnp.concatenate([xd_ref[...], xs_ref[...], ee_ref[...]], axis=-1)
    o_ref[...] = x @ w_ref[...] + b_ref[...][None, :]


def _msg_mm(xd, xs, ee, w, b):
    F = xd.shape[1]
    Fo = w.shape[1]
    return pl.pallas_call(
        _msg_body,
        grid=(_E // _BE,),
        in_specs=[
            pl.BlockSpec((_BE, F), lambda i: (i, 0)),
            pl.BlockSpec((_BE, F), lambda i: (i, 0)),
            pl.BlockSpec((_BE, F), lambda i: (i, 0)),
            pl.BlockSpec((3 * F, Fo), lambda i: (0, 0)),
            pl.BlockSpec((Fo,), lambda i: (0,)),
        ],
        out_specs=pl.BlockSpec((_BE, Fo), lambda i: (i, 0)),
        out_shape=jax.ShapeDtypeStruct((_E, Fo), jnp.float32),
    )(xd, xs, ee, w, b)


def _post_body(h_ref, s_ref, mn_ref, mx_ref, sq_ref, degc_ref, amp_ref,
               att_ref, has_ref, wp_ref, bp_ref, wl_ref, bl_ref, o_ref):
    degc = degc_ref[...]
    has = has_ref[...] > 0.5
    mean = s_ref[...] / degc
    mn = jnp.where(has, mn_ref[...], 0.0)
    mx = jnp.where(has, mx_ref[...], 0.0)
    mean2 = sq_ref[...] / degc
    std = jnp.sqrt(jnp.maximum(mean2 - mean * mean, 0.0) + 1e-5)
    agg = jnp.concatenate([mean, mn, mx, std], axis=-1)
    sc = jnp.concatenate([agg, agg * amp_ref[...], agg * att_ref[...]], axis=-1)
    x = jnp.concatenate([h_ref[...], sc], axis=-1)
    o = x @ wp_ref[...] + bp_ref[...][None, :]
    o = o @ wl_ref[...] + bl_ref[...][None, :]
    o_ref[...] = jnp.where(o > 0, o, 0.0)


def _post_mm(h, s, mn, mx, sq, degc, amp, att, has, wp, bp, wl, bl):
    Dh = h.shape[1]
    F = s.shape[1]
    O = wp.shape[1]
    col = lambda i: (i, 0)
    return pl.pallas_call(
        _post_body,
        grid=(_N // _BN,),
        in_specs=[
            pl.BlockSpec((_BN, Dh), col),
            pl.BlockSpec((_BN, F), col),
            pl.BlockSpec((_BN, F), col),
            pl.BlockSpec((_BN, F), col),
            pl.BlockSpec((_BN, F), col),
            pl.BlockSpec((_BN, 1), col),
            pl.BlockSpec((_BN, 1), col),
            pl.BlockSpec((_BN, 1), col),
            pl.BlockSpec((_BN, 1), col),
            pl.BlockSpec((13 * F, O), lambda i: (0, 0)),
            pl.BlockSpec((O,), lambda i: (0,)),
            pl.BlockSpec((O, O), lambda i: (0, 0)),
            pl.BlockSpec((O,), lambda i: (0,)),
        ],
        out_specs=pl.BlockSpec((_BN, O), col),
        out_shape=jax.ShapeDtypeStruct((_N, O), jnp.float32),
    )(h, s, mn, mx, sq, degc, amp, att, has, wp, bp, wl, bl)


def _head_body(hn_ref, batch_ref, Wd1_ref, bd1_ref, gamma_ref, beta_ref,
               Wd2_ref, bd2_ref, o_ref):
    h = hn_ref[...]
    b = batch_ref[...]
    gids = lax.broadcasted_iota(jnp.int32, (_NG, _N), 0)
    P = (gids == b[None, :]).astype(jnp.float32)
    cnt = jnp.maximum(jnp.sum(P, axis=1), 1.0)
    g = lax.dot(P, h, precision=lax.Precision.HIGHEST) / cnt[:, None]
    z = g @ Wd1_ref[...] + bd1_ref[...][None, :]
    mu = jnp.mean(z, axis=0, keepdims=True)
    var = jnp.mean((z - mu) ** 2, axis=0, keepdims=True)
    z = (z - mu) / jnp.sqrt(var + 1e-5) * gamma_ref[...][None, :] + beta_ref[...][None, :]
    z = jnp.where(z > 0, z, 0.1 * z)
    o_ref[...] = z @ Wd2_ref[...] + bd2_ref[...][None, :]


def _head(h, batch, Wd1, bd1, gamma, beta, Wd2, bd2):
    return pl.pallas_call(
        _head_body,
        out_shape=jax.ShapeDtypeStruct((_NG, Wd2.shape[1]), jnp.float32),
    )(h, batch, Wd1, bd1, gamma, beta, Wd2, bd2)


def kernel(node_attr, edge_index, edge_attr, batch,
           We1, be1, Wpre1, bpre1, Wpost1, bpost1, Wlin1, blin1,
           We2, be2, Wpre2, bpre2, Wpost2, bpost2, Wlin2, blin2,
           Wd1, bd1, gamma, beta, Wd2, bd2):
    src, dst = edge_index[0], edge_index[1]
    deg = jax.ops.segment_sum(jnp.ones((_E,), jnp.float32), dst, num_segments=_N)
    degc = jnp.maximum(deg, 1.0)
    amp = jnp.log(degc + 1.0) / _AVG_LOG
    att = _AVG_LOG / jnp.log(degc + 1.0)
    has = (deg > 0).astype(jnp.float32)
    degc2 = degc[:, None]
    amp2 = amp[:, None]
    att2 = att[:, None]
    has2 = has[:, None]

    def layer(h, gn, We, be, Wpre, bpre, Wpost, bpost, Wlin, blin):
        ee = _edge_mm(edge_attr, We, be)
        xd = gn(dst, h)
        xs = gn(src, h)
        mp = _msg_mm(xd, xs, ee, Wpre, bpre)
        s = jax.ops.segment_sum(mp, dst, num_segments=_N)
        mn = jax.ops.segment_min(mp, dst, num_segments=_N)
        mx = jax.ops.segment_max(mp, dst, num_segments=_N)
        sq = jax.ops.segment_sum(mp * mp, dst, num_segments=_N)
        return _post_mm(h, s, mn, mx, sq, degc2, amp2, att2, has2,
                        Wpost, bpost, Wlin, blin)

    h = layer(node_attr, _g128n, We1, be1, Wpre1, bpre1, Wpost1, bpost1,
              Wlin1, blin1)
    h = layer(h, _g256n, We2, be2, Wpre2, bpre2, Wpost2, bpost2,
              Wlin2, blin2)
    return _head(h, batch, Wd1, bd1, gamma, beta, Wd2, bd2)
